# baseline (speedup 1.0000x reference)
import functools

import jax
import jax.numpy as jnp
from jax import lax
from jax.experimental import pallas as pl
from jax.experimental.pallas import tpu as pltpu
from jax.experimental.pallas import tpu_sc as plsc

C = 128
NC = 2
NS = 16
MM_BLK = 512

_MESH = plsc.VectorSubcoreMesh(core_axis_name="c", subcore_axis_name="s",
                               num_cores=NC, num_subcores=NS)


def _lin_body(ea1_ref, ea2_ref, w1_ref, b1_ref, w2_ref, b2_ref, x1_ref, x2_ref):
    a1 = ea1_ref[...].reshape(MM_BLK, 1)
    a2 = ea2_ref[...].reshape(MM_BLK, 1)
    x1_ref[...] = jnp.maximum(a1 * w1_ref[...] + b1_ref[...][None, :], 0.0)
    x2_ref[...] = jnp.maximum(a2 * w2_ref[...] + b2_ref[...][None, :], 0.0)


def _initial_linear(ea1, ea2, W1, b1, W2, b2, e_pad):
    e1 = ea1.shape[0]
    grid = (e1 // MM_BLK,)
    return pl.pallas_call(
        _lin_body,
        grid=grid,
        in_specs=[
            pl.BlockSpec((MM_BLK,), lambda i: (i,)),
            pl.BlockSpec((MM_BLK,), lambda i: (i,)),
            pl.BlockSpec(W1.shape, lambda i: (0, 0)),
            pl.BlockSpec(b1.shape, lambda i: (0,)),
            pl.BlockSpec(W2.shape, lambda i: (0, 0)),
            pl.BlockSpec(b2.shape, lambda i: (0,)),
        ],
        out_specs=[
            pl.BlockSpec((MM_BLK, C), lambda i: (i, 0)),
            pl.BlockSpec((MM_BLK, C), lambda i: (i, 0)),
        ],
        out_shape=[
            jax.ShapeDtypeStruct((e_pad, C), jnp.float32),
            jax.ShapeDtypeStruct((e_pad, C), jnp.float32),
        ],
    )(ea1[:, 0], ea2[:, 0], W1, b1, W2, b2)


def _mm_body(y1_ref, y2_ref, w1_ref, b1_ref, w2_ref, b2_ref, h1_ref, h2_ref):
    h1_ref[...] = jnp.maximum(
        jnp.dot(y1_ref[...], w1_ref[...], preferred_element_type=jnp.float32)
        + b1_ref[...][None, :], 0.0)
    h2_ref[...] = jnp.maximum(
        jnp.dot(y2_ref[...], w2_ref[...], preferred_element_type=jnp.float32)
        + b2_ref[...][None, :], 0.0)


def _layer_matmul(y1, y2, W1, b1, W2, b2, e1, e_pad):
    grid = (e1 // MM_BLK,)
    return pl.pallas_call(
        _mm_body,
        grid=grid,
        in_specs=[
            pl.BlockSpec((MM_BLK, C), lambda i: (i, 0)),
            pl.BlockSpec((MM_BLK, C), lambda i: (i, 0)),
            pl.BlockSpec(W1.shape, lambda i: (0, 0)),
            pl.BlockSpec(b1.shape, lambda i: (0,)),
            pl.BlockSpec(W2.shape, lambda i: (0, 0)),
            pl.BlockSpec(b2.shape, lambda i: (0,)),
        ],
        out_specs=[
            pl.BlockSpec((MM_BLK, C), lambda i: (i, 0)),
            pl.BlockSpec((MM_BLK, C), lambda i: (i, 0)),
        ],
        out_shape=[
            jax.ShapeDtypeStruct((e_pad, C), jnp.float32),
            jax.ShapeDtypeStruct((e_pad, C), jnp.float32),
        ],
    )(y1, y2, W1, b1, W2, b2)


def _post_body(p_ref, d_ref, w1_ref, b1_ref, w2_ref, b2_ref, out_ref):
    n = p_ref.shape[0] // 2
    pooled = p_ref[:n] + p_ref[n:] + d_ref[0, 0]
    h = jnp.maximum(
        jnp.dot(pooled, w1_ref[...], preferred_element_type=jnp.float32)
        + b1_ref[...][None, :], 0.0)
    out_ref[...] = (jnp.dot(h, w2_ref[...], preferred_element_type=jnp.float32)
                    + b2_ref[...][None, :])[:, 0]


def _post_mlp(P, delta, post_W1, post_b1, post_W2, post_b2):
    n = P.shape[0] // 2
    return pl.pallas_call(
        _post_body,
        out_shape=jax.ShapeDtypeStruct((n,), jnp.float32),
    )(P, delta, post_W1, post_b1, post_W2, post_b2)


def _sym_one(h_hbm, x_hbm, inv_hbm, xo_hbm, invb, invb_t, hb, hgb, xb, sem, s,
             rows_per_tile):
    tile_lo = s * rows_per_tile
    nfull = rows_per_tile // 128
    tail = rows_per_tile - nfull * 128

    semi, semh, semx, semg = sem

    def do_chunk(r0, n, ib):
        ci = pltpu.async_copy(inv_hbm.at[pl.ds(r0, n)], ib, semi)
        ch = pltpu.async_copy(h_hbm.at[pl.ds(r0, n)], hb.at[pl.ds(0, n)], semh)
        cx = pltpu.async_copy(x_hbm.at[pl.ds(r0, n)], xb.at[pl.ds(0, n)], semx)
        ci.wait()
        cg = pltpu.async_copy(h_hbm.at[ib], hgb.at[pl.ds(0, n)], semg)
        ch.wait()
        cx.wait()
        cg.wait()

        def row_body(r, carry):
            for v in range(C // 16):
                sl = pl.ds(v * 16, 16)
                hv = hb[r, sl]
                gv = hgb[r, sl]
                xv = xb[r, sl]
                hb[r, sl] = xv + 0.5 * (hv + gv)
            return carry

        lax.fori_loop(0, n, row_body, 0)
        pltpu.sync_copy(hb.at[pl.ds(0, n)], xo_hbm.at[pl.ds(r0, n)])

    def chunk_body(k, carry):
        do_chunk(tile_lo + k * 128, 128, invb)
        return carry

    lax.fori_loop(0, nfull, chunk_body, 0)
    if tail:
        do_chunk(tile_lo + nfull * 128, tail, invb_t)


def _make_sym(e1, e_pad):
    rows_per_tile = e1 // NS

    @functools.partial(
        pl.kernel,
        mesh=_MESH,
        out_type=[
            jax.ShapeDtypeStruct((e_pad, C), jnp.float32),
            jax.ShapeDtypeStruct((e_pad, C), jnp.float32),
        ],
        scratch_types=[
            pltpu.VMEM((128,), jnp.int32),
            pltpu.VMEM((32,), jnp.int32),
            pltpu.VMEM((128, C), jnp.float32),
            pltpu.VMEM((128, C), jnp.float32),
            pltpu.VMEM((128, C), jnp.float32),
            (pltpu.SemaphoreType.DMA, pltpu.SemaphoreType.DMA,
             pltpu.SemaphoreType.DMA, pltpu.SemaphoreType.DMA),
        ],
    )
    def sym(h1, h2, x1, x2, inv1, inv2, xo1, xo2,
            invb, invb_t, hb, hgb, xb, sem):
        c = lax.axis_index("c")
        s = lax.axis_index("s")

        @pl.when(c == 0)
        def _():
            _sym_one(h1, x1, inv1, xo1, invb, invb_t, hb, hgb, xb, sem, s,
                     rows_per_tile)

        @pl.when(c == 1)
        def _():
            _sym_one(h2, x2, inv2, xo2, invb, invb_t, hb, hgb, xb, sem, s,
                     rows_per_tile)

    return sym


def _pool_one(x_hbm, nd_hbm, shared, xb, nb, nb_t, sem, s, rows_per_tile):
    tile_lo = s * rows_per_tile
    nfull = rows_per_tile // 128
    tail = rows_per_tile - nfull * 128

    semn, semx, sems = sem

    def do_chunk(r0, n, ib):
        cn = pltpu.async_copy(nd_hbm.at[pl.ds(r0, n)], ib, semn)
        cx = pltpu.async_copy(x_hbm.at[pl.ds(r0, n)], xb.at[pl.ds(0, n)], semx)
        cn.wait()
        cx.wait()
        pltpu.async_copy(xb.at[pl.ds(0, n)], shared.at[ib], sems,
                         add=True).wait()

    def chunk_body(k, carry):
        do_chunk(tile_lo + k * 128, 128, nb)
        return carry

    lax.fori_loop(0, nfull, chunk_body, 0)
    if tail:
        do_chunk(tile_lo + nfull * 128, tail, nb_t)


def _stripe_plan(n_nodes):
    big = ((n_nodes // NS) + 7) // 8 * 8
    last = n_nodes - big * (NS - 1)
    return big, last


def _chunk_sizes(total):
    out = []
    while total > 0:
        sz = min(128, total)
        out.append(sz)
        total -= sz
    return out


def _make_pool(e1, n_nodes):
    rows_per_tile = e1 // NS
    big, last = _stripe_plan(n_nodes)

    @functools.partial(
        pl.kernel,
        mesh=_MESH,
        out_type=jax.ShapeDtypeStruct((NC * n_nodes, C), jnp.float32),
        scratch_types=[
            pltpu.VMEM_SHARED((n_nodes, C), jnp.float32),
            pltpu.VMEM((128, C), jnp.float32),
            pltpu.VMEM((128,), jnp.int32),
            pltpu.VMEM((32,), jnp.int32),
            (pltpu.SemaphoreType.DMA, pltpu.SemaphoreType.DMA,
             pltpu.SemaphoreType.DMA),
        ],
    )
    def pool(x1, x2, nd1, nd2, out, shared, xb, nb, nb_t, sem):
        c = lax.axis_index("c")
        s = lax.axis_index("s")

        def zrow(r, carry):
            for v in range(C // 16):
                xb[r, pl.ds(v * 16, 16)] = jnp.zeros((16,), jnp.float32)
            return carry

        lax.fori_loop(0, 128, zrow, 0)

        def zero_stripe(lo, total):
            off = 0
            for sz in _chunk_sizes(total):
                pltpu.sync_copy(xb.at[pl.ds(0, sz)],
                                shared.at[pl.ds(lo + off, sz)])
                off += sz

        @pl.when(s < NS - 1)
        def _():
            zero_stripe(s * big, big)

        @pl.when(s == NS - 1)
        def _():
            zero_stripe((NS - 1) * big, last)

        plsc.subcore_barrier()

        @pl.when(c == 0)
        def _():
            _pool_one(x1, nd1, shared, xb, nb, nb_t, sem, s, rows_per_tile)

        @pl.when(c == 1)
        def _():
            _pool_one(x2, nd2, shared, xb, nb, nb_t, sem, s, rows_per_tile)

        plsc.subcore_barrier()

        def copy_stripe(lo, total):
            off = 0
            for sz in _chunk_sizes(total):
                rows = pl.ds(lo + off, sz)
                pltpu.sync_copy(shared.at[rows], xb.at[pl.ds(0, sz)])
                pltpu.sync_copy(xb.at[pl.ds(0, sz)],
                                out.at[pl.ds(c * n_nodes + lo + off, sz)])
                off += sz

        @pl.when(s < NS - 1)
        def _():
            copy_stripe(s * big, big)

        @pl.when(s == NS - 1)
        def _():
            copy_stripe((NS - 1) * big, last)

    return pool


def kernel(edge_attr, edge_attr2, triangle_1_1_1, triangle_1_1_2, triangle_1_2_2,
           triangle_2_2_2, inverse_edge_1, inverse_edge_2, edge_index, edge_index2,
           num_nodes, lin_W1, lin_b1, lin_W2, lin_b2, ker_W1, ker_b1, ker_W2, ker_b2,
           post_W1, post_b1, post_W2, post_b2):
    e1 = edge_attr.shape[0]
    e2 = edge_attr2.shape[0]
    n_nodes = 10000
    BR = 16256
    nblk = -(-e1 // BR)
    e_pad = nblk * BR

    x1, x2 = _initial_linear(edge_attr, edge_attr2, lin_W1, lin_b1,
                             lin_W2, lin_b2, e_pad)

    sym = _make_sym(e1, e_pad)

    def layer(carry, w):
        x1, x2 = carry
        w1, b1, w2, b2 = w
        xs1 = x1[:e1]
        xs2 = x2[:e2]
        m1 = jax.ops.segment_sum(xs1[triangle_1_1_1[1]] * xs1[triangle_1_1_1[2]], triangle_1_1_1[0], num_segments=e1)
        m1 = m1 + jax.ops.segment_sum(xs1[triangle_1_1_2[1]] * xs2[triangle_1_1_2[2]], triangle_1_1_2[0], num_segments=e1)
        m1 = m1 + jax.ops.segment_sum(xs2[triangle_1_2_2[1]] * xs2[triangle_1_2_2[2]], triangle_1_2_2[0], num_segments=e1)
        m2 = jax.ops.segment_sum(xs1[triangle_1_1_2[1]] * xs1[triangle_1_1_2[2]], triangle_1_1_2[0], num_segments=e2)
        m2 = m2 + jax.ops.segment_sum(xs1[triangle_1_2_2[1]] * xs2[triangle_1_2_2[2]], triangle_1_2_2[0], num_segments=e2)
        m2 = m2 + jax.ops.segment_sum(xs2[triangle_2_2_2[1]] * xs2[triangle_2_2_2[2]], triangle_2_2_2[0], num_segments=e2)
        y1 = jnp.concatenate([xs1 + m1, jnp.zeros((e_pad - e1, C), jnp.float32)], 0)
        y2 = jnp.concatenate([xs2 + m2, jnp.zeros((e_pad - e2, C), jnp.float32)], 0)
        h1, h2 = _layer_matmul(y1, y2, w1, b1, w2, b2, e1, e_pad)
        x1, x2 = sym(h1, h2, x1, x2, inverse_edge_1, inverse_edge_2)
        return (x1, x2), None

    (x1, x2), _ = lax.scan(layer, (x1, x2),
                           (ker_W1, ker_b1, ker_W2, ker_b2))

    pool = _make_pool(e1, n_nodes)
    P = pool(x1, x2, edge_index[1], edge_index2[1])
    delta = jnp.asarray(num_nodes - n_nodes, jnp.float32).reshape(1, 1)
    return _post_mlp(P, delta, post_W1, post_b1, post_W2, post_b2)
